# initial kernel scaffold (unmeasured)
import jax
import jax.numpy as jnp
from jax import lax
from jax.experimental import pallas as pl
from jax.experimental.pallas import tpu as pltpu

N_DEV = 8
M, K, N = 4096, 4096, 8192
KS = K // N_DEV
NS = N // N_DEV


def _entry_barrier(d):
    bsem = pltpu.get_barrier_semaphore()
    for j in range(1, N_DEV):
        pl.semaphore_signal(
            bsem,
            inc=1,
            device_id=((d + j) % N_DEV,),
            device_id_type=pl.DeviceIdType.MESH,
        )
    pl.semaphore_wait(bsem, N_DEV - 1)


def _gather_body(xs_ref, ws_ref, xfull_ref, wmine_ref, xsend, xrecv, wsend, wrecv):
    d = lax.axis_index("i")
    _entry_barrier(d)

    xfull_ref[:, pl.ds(d * KS, KS)] = xs_ref[...]
    wmine_ref[pl.ds(d * KS, KS), :] = ws_ref[:, pl.ds(d * NS, NS)]

    rdmas = []
    for j in range(1, N_DEV):
        p = (d + j) % N_DEV
        rx = pltpu.make_async_remote_copy(
            src_ref=xs_ref,
            dst_ref=xfull_ref.at[:, pl.ds(d * KS, KS)],
            send_sem=xsend.at[j - 1],
            recv_sem=xrecv.at[j - 1],
            device_id=(p,),
            device_id_type=pl.DeviceIdType.MESH,
        )
        rx.start()
        rw = pltpu.make_async_remote_copy(
            src_ref=ws_ref.at[:, pl.ds(p * NS, NS)],
            dst_ref=wmine_ref.at[pl.ds(d * KS, KS), :],
            send_sem=wsend.at[j - 1],
            recv_sem=wrecv.at[j - 1],
            device_id=(p,),
            device_id_type=pl.DeviceIdType.MESH,
        )
        rw.start()
        rdmas.append((rx, rw))

    for rx, rw in rdmas:
        rx.wait_send()
        rw.wait_send()
    for rx, rw in rdmas:
        rx.wait_recv()
        rw.wait_recv()


def _epilogue_body(
    y_ref, qfull_ref, amax_ref, qmine, amaxbuf, asend, arecv, qsend, qrecv
):
    d = lax.axis_index("i")
    _entry_barrier(d)

    amax_local = jnp.maximum(jnp.max(y_ref[...]), 0.0)
    amaxbuf[pl.ds(d, 1), :] = jnp.full((1, 128), amax_local, jnp.float32)

    ardmas = []
    for j in range(1, N_DEV):
        p = (d + j) % N_DEV
        ra = pltpu.make_async_remote_copy(
            src_ref=amaxbuf.at[pl.ds(d, 1), :],
            dst_ref=amaxbuf.at[pl.ds(d, 1), :],
            send_sem=asend.at[j - 1],
            recv_sem=arecv.at[j - 1],
            device_id=(p,),
            device_id_type=pl.DeviceIdType.MESH,
        )
        ra.start()
        ardmas.append(ra)
    for ra in ardmas:
        ra.wait_send()
    for ra in ardmas:
        ra.wait_recv()

    amax_g = jnp.max(amaxbuf[...])
    scale = jnp.maximum(amax_g, 1e-30) / 448.0

    qmine[...] = (jnp.maximum(y_ref[...], 0.0) / scale).astype(jnp.float8_e4m3fn)
    qfull_ref[:, pl.ds(d * NS, NS)] = qmine[...]

    qrdmas = []
    for j in range(1, N_DEV):
        p = (d + j) % N_DEV
        rq = pltpu.make_async_remote_copy(
            src_ref=qmine,
            dst_ref=qfull_ref.at[:, pl.ds(d * NS, NS)],
            send_sem=qsend.at[j - 1],
            recv_sem=qrecv.at[j - 1],
            device_id=(p,),
            device_id_type=pl.DeviceIdType.MESH,
        )
        rq.start()
        qrdmas.append(rq)
    for rq in qrdmas:
        rq.wait_send()
    for rq in qrdmas:
        rq.wait_recv()

    amax_ref[...] = amaxbuf[...]


def kernel(x, w_mat):
    xb = x.astype(jnp.bfloat16)
    wb = w_mat.astype(jnp.bfloat16)

    xfull, wmine = pl.pallas_call(
        _gather_body,
        out_shape=[
            jax.ShapeDtypeStruct((M, K), jnp.bfloat16),
            jax.ShapeDtypeStruct((K, NS), jnp.bfloat16),
        ],
        in_specs=[
            pl.BlockSpec(memory_space=pltpu.VMEM),
            pl.BlockSpec(memory_space=pltpu.VMEM),
        ],
        out_specs=[
            pl.BlockSpec(memory_space=pltpu.VMEM),
            pl.BlockSpec(memory_space=pltpu.VMEM),
        ],
        scratch_shapes=[
            pltpu.SemaphoreType.DMA((N_DEV - 1,)),
            pltpu.SemaphoreType.DMA((N_DEV - 1,)),
            pltpu.SemaphoreType.DMA((N_DEV - 1,)),
            pltpu.SemaphoreType.DMA((N_DEV - 1,)),
        ],
        compiler_params=pltpu.CompilerParams(collective_id=0),
    )(xb, wb)

    y = jnp.dot(xfull, wmine, preferred_element_type=jnp.float32)

    qfull, amaxes = pl.pallas_call(
        _epilogue_body,
        out_shape=[
            jax.ShapeDtypeStruct((M, N), jnp.float8_e4m3fn),
            jax.ShapeDtypeStruct((N_DEV, 128), jnp.float32),
        ],
        in_specs=[pl.BlockSpec(memory_space=pltpu.VMEM)],
        out_specs=[
            pl.BlockSpec(memory_space=pltpu.VMEM),
            pl.BlockSpec(memory_space=pltpu.VMEM),
        ],
        scratch_shapes=[
            pltpu.VMEM((M, NS), jnp.float8_e4m3fn),
            pltpu.VMEM((N_DEV, 128), jnp.float32),
            pltpu.SemaphoreType.DMA((N_DEV - 1,)),
            pltpu.SemaphoreType.DMA((N_DEV - 1,)),
            pltpu.SemaphoreType.DMA((N_DEV - 1,)),
            pltpu.SemaphoreType.DMA((N_DEV - 1,)),
        ],
        compiler_params=pltpu.CompilerParams(collective_id=1),
    )(y)

    scale = jnp.maximum(jnp.max(amaxes), 1e-30) / 448.0
    return (qfull.astype(jnp.float32) * scale).astype(jnp.bfloat16)


# baseline (device time: 732348 ns/iter reference)
import jax
import jax.numpy as jnp
from jax import lax
from jax.experimental import pallas as pl
from jax.experimental.pallas import tpu as pltpu

N_DEV = 8
M, K, N = 4096, 4096, 8192
KS = K // N_DEV
NS = N // N_DEV


def _entry_barrier(d):
    bsem = pltpu.get_barrier_semaphore()
    for j in range(1, N_DEV):
        pl.semaphore_signal(
            bsem,
            inc=1,
            device_id=((d + j) % N_DEV,),
            device_id_type=pl.DeviceIdType.MESH,
        )
    pl.semaphore_wait(bsem, N_DEV - 1)


def _gather_body(
    xs_ref, ws_ref, xfull_ref, wmine_ref, lsem, xsend, xrecv, wsend, wrecv
):
    d = lax.axis_index("i")
    _entry_barrier(d)

    own = pltpu.make_async_copy(xs_ref, xfull_ref.at[:, pl.ds(d * KS, KS)], lsem)
    own.start()
    wmine_ref[pl.ds(d * KS, KS), :] = ws_ref[:, pl.ds(d * NS, NS)]

    rdmas = []
    for j in range(1, N_DEV):
        p = (d + j) % N_DEV
        rx = pltpu.make_async_remote_copy(
            src_ref=xs_ref,
            dst_ref=xfull_ref.at[:, pl.ds(d * KS, KS)],
            send_sem=xsend.at[j - 1],
            recv_sem=xrecv.at[j - 1],
            device_id=(p,),
            device_id_type=pl.DeviceIdType.MESH,
        )
        rx.start()
        rw = pltpu.make_async_remote_copy(
            src_ref=ws_ref.at[:, pl.ds(p * NS, NS)],
            dst_ref=wmine_ref.at[pl.ds(d * KS, KS), :],
            send_sem=wsend.at[j - 1],
            recv_sem=wrecv.at[j - 1],
            device_id=(p,),
            device_id_type=pl.DeviceIdType.MESH,
        )
        rw.start()
        rdmas.append((rx, rw))

    own.wait()
    for rx, rw in rdmas:
        rx.wait_send()
        rw.wait_send()
    for rx, rw in rdmas:
        rx.wait_recv()
        rw.wait_recv()


def _epilogue_body(
    y_ref, qfull_ref, amax_ref, qmine, amaxbuf, lsem, asend, arecv, qsend, qrecv
):
    d = lax.axis_index("i")
    _entry_barrier(d)

    amax_local = jnp.maximum(jnp.max(y_ref[...]), 0.0)
    amaxbuf[pl.ds(d, 1), :] = jnp.full((1, 128), amax_local, jnp.float32)

    ardmas = []
    for j in range(1, N_DEV):
        p = (d + j) % N_DEV
        ra = pltpu.make_async_remote_copy(
            src_ref=amaxbuf.at[pl.ds(d, 1), :],
            dst_ref=amaxbuf.at[pl.ds(d, 1), :],
            send_sem=asend.at[j - 1],
            recv_sem=arecv.at[j - 1],
            device_id=(p,),
            device_id_type=pl.DeviceIdType.MESH,
        )
        ra.start()
        ardmas.append(ra)
    for ra in ardmas:
        ra.wait_send()
    for ra in ardmas:
        ra.wait_recv()

    amax_g = jnp.max(amaxbuf[...])
    scale = jnp.maximum(amax_g, 1e-30) / 448.0

    qmine[...] = (jnp.maximum(y_ref[...], 0.0) / scale).astype(jnp.float8_e4m3fn)
    own = pltpu.make_async_copy(qmine, qfull_ref.at[:, pl.ds(d * NS, NS)], lsem)
    own.start()

    qrdmas = []
    for j in range(1, N_DEV):
        p = (d + j) % N_DEV
        rq = pltpu.make_async_remote_copy(
            src_ref=qmine,
            dst_ref=qfull_ref.at[:, pl.ds(d * NS, NS)],
            send_sem=qsend.at[j - 1],
            recv_sem=qrecv.at[j - 1],
            device_id=(p,),
            device_id_type=pl.DeviceIdType.MESH,
        )
        rq.start()
        qrdmas.append(rq)
    own.wait()
    for rq in qrdmas:
        rq.wait_send()
    for rq in qrdmas:
        rq.wait_recv()

    amax_ref[...] = amaxbuf[...]


def kernel(x, w_mat):
    xb = x.astype(jnp.bfloat16)
    wb = w_mat.astype(jnp.bfloat16)

    xfull, wmine = pl.pallas_call(
        _gather_body,
        out_shape=[
            jax.ShapeDtypeStruct((M, K), jnp.bfloat16),
            jax.ShapeDtypeStruct((K, NS), jnp.bfloat16),
        ],
        in_specs=[
            pl.BlockSpec(memory_space=pltpu.VMEM),
            pl.BlockSpec(memory_space=pltpu.VMEM),
        ],
        out_specs=[
            pl.BlockSpec(memory_space=pl.ANY),
            pl.BlockSpec(memory_space=pltpu.VMEM),
        ],
        scratch_shapes=[
            pltpu.SemaphoreType.DMA,
            pltpu.SemaphoreType.DMA((N_DEV - 1,)),
            pltpu.SemaphoreType.DMA((N_DEV - 1,)),
            pltpu.SemaphoreType.DMA((N_DEV - 1,)),
            pltpu.SemaphoreType.DMA((N_DEV - 1,)),
        ],
        compiler_params=pltpu.CompilerParams(
            collective_id=0, vmem_limit_bytes=64 * 1024 * 1024
        ),
    )(xb, wb)

    y = jnp.dot(xfull, wmine, preferred_element_type=jnp.float32)

    qfull, amaxes = pl.pallas_call(
        _epilogue_body,
        out_shape=[
            jax.ShapeDtypeStruct((M, N), jnp.float8_e4m3fn),
            jax.ShapeDtypeStruct((N_DEV, 128), jnp.float32),
        ],
        in_specs=[pl.BlockSpec(memory_space=pltpu.VMEM)],
        out_specs=[
            pl.BlockSpec(memory_space=pl.ANY),
            pl.BlockSpec(memory_space=pltpu.VMEM),
        ],
        scratch_shapes=[
            pltpu.VMEM((M, NS), jnp.float8_e4m3fn),
            pltpu.VMEM((N_DEV, 128), jnp.float32),
            pltpu.SemaphoreType.DMA,
            pltpu.SemaphoreType.DMA((N_DEV - 1,)),
            pltpu.SemaphoreType.DMA((N_DEV - 1,)),
            pltpu.SemaphoreType.DMA((N_DEV - 1,)),
            pltpu.SemaphoreType.DMA((N_DEV - 1,)),
        ],
        compiler_params=pltpu.CompilerParams(
            collective_id=1, vmem_limit_bytes=64 * 1024 * 1024
        ),
    )(y)

    scale = jnp.maximum(jnp.max(amaxes), 1e-30) / 448.0
    return (qfull.astype(jnp.float32) * scale).astype(jnp.bfloat16)


# device time: 727466 ns/iter; 1.0067x vs baseline; 1.0067x over previous
import jax
import jax.numpy as jnp
from jax import lax
from jax.experimental import pallas as pl
from jax.experimental.pallas import tpu as pltpu

N_DEV = 8
M, K, N = 4096, 4096, 8192
KS = K // N_DEV
NS = N // N_DEV


def _entry_barrier(d):
    bsem = pltpu.get_barrier_semaphore()
    for j in range(1, N_DEV):
        pl.semaphore_signal(
            bsem,
            inc=1,
            device_id=((d + j) % N_DEV,),
            device_id_type=pl.DeviceIdType.MESH,
        )
    pl.semaphore_wait(bsem, N_DEV - 1)


def _gather_gemm_body(
    xb_ref, wb_ref, y_ref, xg, wmine, xblk, cpsem, xsend, xrecv, wsend, wrecv
):
    d = lax.axis_index("i")
    _entry_barrier(d)

    rdmas = []
    for j in range(1, N_DEV):
        p = (d + j) % N_DEV
        rw = pltpu.make_async_remote_copy(
            src_ref=wb_ref.at[:, pl.ds(p * NS, NS)],
            dst_ref=wmine.at[pl.ds(d * KS, KS), :],
            send_sem=wsend.at[j - 1],
            recv_sem=wrecv.at[j - 1],
            device_id=(p,),
            device_id_type=pl.DeviceIdType.MESH,
        )
        rw.start()
        rx = pltpu.make_async_remote_copy(
            src_ref=xb_ref,
            dst_ref=xg.at[:, pl.ds(d * KS, KS)],
            send_sem=xsend.at[j - 1],
            recv_sem=xrecv.at[j - 1],
            device_id=(p,),
            device_id_type=pl.DeviceIdType.MESH,
        )
        rx.start()
        rdmas.append((rx, rw))

    y_ref[...] = jnp.dot(
        xb_ref[...],
        wb_ref[:, pl.ds(d * NS, NS)],
        preferred_element_type=jnp.float32,
    )

    def stage_copy(j):
        s = (d - j) % N_DEV
        cp = pltpu.make_async_copy(
            xg.at[:, pl.ds(s * KS, KS)], xblk.at[(j - 1) % 2], cpsem.at[(j - 1) % 2]
        )
        cp.start()
        return cp

    rdmas[0][0].wait_recv()
    cp = stage_copy(1)
    for j in range(1, N_DEV):
        cp.wait()
        rdmas[j - 1][1].wait_recv()
        if j < N_DEV - 1:
            rdmas[j][0].wait_recv()
            cp = stage_copy(j + 1)
        s = (d - j) % N_DEV
        y_ref[...] += jnp.dot(
            xblk[(j - 1) % 2],
            wmine[pl.ds(s * KS, KS), :],
            preferred_element_type=jnp.float32,
        )

    for rx, rw in rdmas:
        rx.wait_send()
        rw.wait_send()


def _epilogue_body(
    y_ref,
    out_ref,
    qg,

    qmine,
    qblk,
    stage,
    amaxbuf,
    qcpsem,
    stsem,
    asend,
    arecv,
    qsend,
    qrecv,
):
    d = lax.axis_index("i")
    _entry_barrier(d)

    amax_local = jnp.maximum(jnp.max(y_ref[...]), 0.0)
    amaxbuf[pl.ds(d, 1), :] = jnp.full((1, 128), amax_local, jnp.float32)

    ardmas = []
    for j in range(1, N_DEV):
        p = (d + j) % N_DEV
        ra = pltpu.make_async_remote_copy(
            src_ref=amaxbuf.at[pl.ds(d, 1), :],
            dst_ref=amaxbuf.at[pl.ds(d, 1), :],
            send_sem=asend.at[j - 1],
            recv_sem=arecv.at[j - 1],
            device_id=(p,),
            device_id_type=pl.DeviceIdType.MESH,
        )
        ra.start()
        ardmas.append(ra)
    for ra in ardmas:
        ra.wait_send()
    for ra in ardmas:
        ra.wait_recv()

    amax_g = jnp.max(amaxbuf[...])
    scale = jnp.maximum(amax_g, 1e-30) / 448.0

    qmine[...] = (jnp.maximum(y_ref[...], 0.0) / scale).astype(jnp.float8_e4m3fn)

    qrdmas = []
    for j in range(1, N_DEV):
        p = (d + j) % N_DEV
        rq = pltpu.make_async_remote_copy(
            src_ref=qmine,
            dst_ref=qg.at[:, pl.ds(d * NS, NS)],
            send_sem=qsend.at[j - 1],
            recv_sem=qrecv.at[j - 1],
            device_id=(p,),
            device_id_type=pl.DeviceIdType.MESH,
        )
        rq.start()
        qrdmas.append(rq)

    out_copies = []
    for j in range(0, N_DEV):
        slot = j % 2
        s = (d - j) % N_DEV
        if j == 0:
            qv = qmine[...]
        else:
            qrdmas[j - 1].wait_recv()
            cp = pltpu.make_async_copy(
                qg.at[:, pl.ds(s * NS, NS)], qblk.at[slot], qcpsem.at[slot]
            )
            cp.start()
            cp.wait()
            qv = qblk[slot]
        if j >= 2:
            out_copies[j - 2].wait()
        stage[slot, :, :] = (qv.astype(jnp.float32) * scale).astype(jnp.bfloat16)
        ocp = pltpu.make_async_copy(
            stage.at[slot], out_ref.at[:, pl.ds(s * NS, NS)], stsem.at[slot]
        )
        ocp.start()
        out_copies.append(ocp)
    out_copies[-2].wait()
    out_copies[-1].wait()

    for rq in qrdmas:
        rq.wait_send()


def kernel(x, w_mat):
    xb = x.astype(jnp.bfloat16)
    wb = w_mat.astype(jnp.bfloat16)

    y, _xg = pl.pallas_call(
        _gather_gemm_body,
        out_shape=[
            jax.ShapeDtypeStruct((M, NS), jnp.float32),
            jax.ShapeDtypeStruct((M, K), jnp.bfloat16),
        ],
        in_specs=[
            pl.BlockSpec(memory_space=pltpu.VMEM),
            pl.BlockSpec(memory_space=pltpu.VMEM),
        ],
        out_specs=[
            pl.BlockSpec(memory_space=pltpu.VMEM),
            pl.BlockSpec(memory_space=pl.ANY),
        ],
        scratch_shapes=[
            pltpu.VMEM((K, NS), jnp.bfloat16),
            pltpu.VMEM((2, M, KS), jnp.bfloat16),
            pltpu.SemaphoreType.DMA((2,)),
            pltpu.SemaphoreType.DMA((N_DEV - 1,)),
            pltpu.SemaphoreType.DMA((N_DEV - 1,)),
            pltpu.SemaphoreType.DMA((N_DEV - 1,)),
            pltpu.SemaphoreType.DMA((N_DEV - 1,)),
        ],
        compiler_params=pltpu.CompilerParams(
            collective_id=0, vmem_limit_bytes=60 * 1024 * 1024
        ),
    )(xb, wb)

    out, _qg = pl.pallas_call(
        _epilogue_body,
        out_shape=[
            jax.ShapeDtypeStruct((M, N), jnp.bfloat16),
            jax.ShapeDtypeStruct((M, N), jnp.float8_e4m3fn),
        ],
        in_specs=[pl.BlockSpec(memory_space=pltpu.VMEM)],
        out_specs=[
            pl.BlockSpec(memory_space=pl.ANY),
            pl.BlockSpec(memory_space=pl.ANY),
        ],
        scratch_shapes=[
            pltpu.VMEM((M, NS), jnp.float8_e4m3fn),
            pltpu.VMEM((2, M, NS), jnp.float8_e4m3fn),
            pltpu.VMEM((2, M, NS), jnp.bfloat16),
            pltpu.VMEM((N_DEV, 128), jnp.float32),
            pltpu.SemaphoreType.DMA((2,)),
            pltpu.SemaphoreType.DMA((2,)),
            pltpu.SemaphoreType.DMA((N_DEV - 1,)),
            pltpu.SemaphoreType.DMA((N_DEV - 1,)),
            pltpu.SemaphoreType.DMA((N_DEV - 1,)),
            pltpu.SemaphoreType.DMA((N_DEV - 1,)),
        ],
        compiler_params=pltpu.CompilerParams(
            collective_id=1, vmem_limit_bytes=60 * 1024 * 1024
        ),
    )(y)
    return out


# device time: 711402 ns/iter; 1.0294x vs baseline; 1.0226x over previous
import jax
import jax.numpy as jnp
from jax import lax
from jax.experimental import pallas as pl
from jax.experimental.pallas import tpu as pltpu

N_DEV = 8
M, K, N = 4096, 4096, 8192
KS = K // N_DEV
NS = N // N_DEV
HS = NS // 2


def _fused_body(
    xb_ref,
    wb_ref,
    out_ref,
    y,
    xg,
    qg,
    wmine,
    xblk,
    wown,
    qmine,
    qblk,
    stage,
    amaxbuf,
    cpsem,
    qcpsem,
    stsem,
    xsend,
    xrecv,
    wsend,
    wrecv,
    asend,
    arecv,
    qsend,
    qrecv,
):
    d = lax.axis_index("i")

    bsem = pltpu.get_barrier_semaphore()
    for j in range(1, N_DEV):
        pl.semaphore_signal(
            bsem,
            inc=1,
            device_id=((d + j) % N_DEV,),
            device_id_type=pl.DeviceIdType.MESH,
        )
    pl.semaphore_wait(bsem, N_DEV - 1)

    rdmas = []
    for j in range(1, N_DEV):
        p = (d + j) % N_DEV
        rw = pltpu.make_async_remote_copy(
            src_ref=wb_ref.at[:, pl.ds(p * NS, NS)],
            dst_ref=wmine.at[pl.ds(d * KS, KS), :],
            send_sem=wsend.at[j - 1],
            recv_sem=wrecv.at[j - 1],
            device_id=(p,),
            device_id_type=pl.DeviceIdType.MESH,
        )
        rw.start()
        rx = pltpu.make_async_remote_copy(
            src_ref=xb_ref,
            dst_ref=xg.at[:, pl.ds(d * KS, KS)],
            send_sem=xsend.at[j - 1],
            recv_sem=xrecv.at[j - 1],
            device_id=(p,),
            device_id_type=pl.DeviceIdType.MESH,
        )
        rx.start()
        rdmas.append((rx, rw))

    cpx = pltpu.make_async_copy(xb_ref, xblk, cpsem)
    cpx.start()
    cpw = pltpu.make_async_copy(wb_ref.at[:, pl.ds(d * NS, NS)], wown, qcpsem)
    cpw.start()
    cpx.wait()
    cpw.wait()
    y[...] = jnp.dot(xblk[...], wown[...], preferred_element_type=jnp.float32)

    for j in range(1, N_DEV):
        s = (d - j) % N_DEV
        rdmas[j - 1][0].wait_recv()
        cp = pltpu.make_async_copy(xg.at[:, pl.ds(s * KS, KS)], xblk, cpsem)
        cp.start()
        cp.wait()
        rdmas[j - 1][1].wait_recv()
        y[...] += jnp.dot(
            xblk[...],
            wmine[pl.ds(s * KS, KS), :],
            preferred_element_type=jnp.float32,
        )

    amax_local = jnp.maximum(jnp.max(y[...]), 0.0)
    amaxbuf[pl.ds(d, 1), :] = jnp.full((1, 128), amax_local, jnp.float32)
    ardmas = []
    for j in range(1, N_DEV):
        p = (d + j) % N_DEV
        ra = pltpu.make_async_remote_copy(
            src_ref=amaxbuf.at[pl.ds(d, 1), :],
            dst_ref=amaxbuf.at[pl.ds(d, 1), :],
            send_sem=asend.at[j - 1],
            recv_sem=arecv.at[j - 1],
            device_id=(p,),
            device_id_type=pl.DeviceIdType.MESH,
        )
        ra.start()
        ardmas.append(ra)
    for ra in ardmas:
        ra.wait_recv()

    amax_g = jnp.max(amaxbuf[...])
    scale = jnp.maximum(amax_g, 1e-30) / 448.0

    qmine[...] = (jnp.maximum(y[...], 0.0) / scale).astype(jnp.float8_e4m3fn)
    qrdmas = []
    for j in range(1, N_DEV):
        p = (d + j) % N_DEV
        rq = pltpu.make_async_remote_copy(
            src_ref=qmine,
            dst_ref=qg.at[:, pl.ds(d * NS, NS)],
            send_sem=qsend.at[j - 1],
            recv_sem=qrecv.at[j - 1],
            device_id=(p,),
            device_id_type=pl.DeviceIdType.MESH,
        )
        rq.start()
        qrdmas.append(rq)

    out_copies = []
    idx = 0
    for j in range(0, N_DEV):
        s = (d - j) % N_DEV
        if j == 0:
            qsrc = qmine
        else:
            qrdmas[j - 1].wait_recv()
            cp = pltpu.make_async_copy(
                qg.at[:, pl.ds(s * NS, NS)], qblk, qcpsem
            )
            cp.start()
            cp.wait()
            qsrc = qblk
        for h in range(2):
            slot = idx % 2
            if idx >= 2:
                out_copies[idx - 2].wait()
            stage[slot, :, :] = (
                qsrc[:, h * HS : (h + 1) * HS].astype(jnp.float32) * scale
            ).astype(jnp.bfloat16)
            ocp = pltpu.make_async_copy(
                stage.at[slot],
                out_ref.at[:, pl.ds(s * NS + h * HS, HS)],
                stsem.at[slot],
            )
            ocp.start()
            out_copies.append(ocp)
            idx += 1
    out_copies[-2].wait()
    out_copies[-1].wait()

    for rx, rw in rdmas:
        rx.wait_send()
        rw.wait_send()
    for ra in ardmas:
        ra.wait_send()
    for rq in qrdmas:
        rq.wait_send()


def kernel(x, w_mat):
    xb = x.astype(jnp.bfloat16)
    wb = w_mat.astype(jnp.bfloat16)

    out, _y, _xg, _qg = pl.pallas_call(
        _fused_body,
        out_shape=[
            jax.ShapeDtypeStruct((M, N), jnp.bfloat16),
            jax.ShapeDtypeStruct((M, NS), jnp.float32),
            jax.ShapeDtypeStruct((M, K), jnp.bfloat16),
            jax.ShapeDtypeStruct((M, N), jnp.float8_e4m3fn),
        ],
        in_specs=[
            pl.BlockSpec(memory_space=pl.ANY),
            pl.BlockSpec(memory_space=pl.ANY),
        ],
        out_specs=[
            pl.BlockSpec(memory_space=pl.ANY),
            pl.BlockSpec(memory_space=pltpu.VMEM),
            pl.BlockSpec(memory_space=pl.ANY),
            pl.BlockSpec(memory_space=pl.ANY),
        ],
        scratch_shapes=[
            pltpu.VMEM((K, NS), jnp.bfloat16),
            pltpu.VMEM((M, KS), jnp.bfloat16),
            pltpu.VMEM((KS, NS), jnp.bfloat16),
            pltpu.VMEM((M, NS), jnp.float8_e4m3fn),
            pltpu.VMEM((M, NS), jnp.float8_e4m3fn),
            pltpu.VMEM((2, M, HS), jnp.bfloat16),
            pltpu.VMEM((N_DEV, 128), jnp.float32),
            pltpu.SemaphoreType.DMA,
            pltpu.SemaphoreType.DMA,
            pltpu.SemaphoreType.DMA((2,)),
            pltpu.SemaphoreType.DMA((N_DEV - 1,)),
            pltpu.SemaphoreType.DMA((N_DEV - 1,)),
            pltpu.SemaphoreType.DMA((N_DEV - 1,)),
            pltpu.SemaphoreType.DMA((N_DEV - 1,)),
            pltpu.SemaphoreType.DMA((N_DEV - 1,)),
            pltpu.SemaphoreType.DMA((N_DEV - 1,)),
            pltpu.SemaphoreType.DMA((N_DEV - 1,)),
            pltpu.SemaphoreType.DMA((N_DEV - 1,)),
        ],
        compiler_params=pltpu.CompilerParams(
            collective_id=0, vmem_limit_bytes=63 * 1024 * 1024
        ),
    )(xb, wb)
    return out


# device time: 643586 ns/iter; 1.1379x vs baseline; 1.1054x over previous
import jax
import jax.numpy as jnp
from jax import lax
from jax.experimental import pallas as pl
from jax.experimental.pallas import tpu as pltpu

N_DEV = 8
M, K, N = 4096, 4096, 8192
KS = K // N_DEV
PM, PN = 2, 4
MT = M // PM
NT = N // PN
HT = NT // 2


def _fused_body(
    xb_ref,
    wb_ref,
    out_ref,
    y,
    xg,
    wg,
    qg,
    xblk,
    wblk,
    qmine,
    qblk,
    stage,
    amaxbuf,
    cpxsem,
    cpwsem,
    stsem,
    xsend,
    xrecv,
    wsend,
    wrecv,
    asend,
    arecv,
    qsend,
    qrecv,
):
    d = lax.axis_index("i")
    r = d // PN
    c = d % PN

    bsem = pltpu.get_barrier_semaphore()
    for j in range(1, N_DEV):
        pl.semaphore_signal(
            bsem,
            inc=1,
            device_id=((d + j) % N_DEV,),
            device_id_type=pl.DeviceIdType.MESH,
        )
    pl.semaphore_wait(bsem, N_DEV - 1)

    rdmas = []
    for j in range(1, N_DEV):
        p = (d + j) % N_DEV
        rp = p // PN
        cp_ = p % PN
        rw = pltpu.make_async_remote_copy(
            src_ref=wb_ref.at[:, pl.ds(cp_ * NT, NT)],
            dst_ref=wg.at[pl.ds(d * KS, KS), :],
            send_sem=wsend.at[j - 1],
            recv_sem=wrecv.at[j - 1],
            device_id=(p,),
            device_id_type=pl.DeviceIdType.MESH,
        )
        rw.start()
        rx = pltpu.make_async_remote_copy(
            src_ref=xb_ref.at[pl.ds(rp * MT, MT), :],
            dst_ref=xg.at[:, pl.ds(d * KS, KS)],
            send_sem=xsend.at[j - 1],
            recv_sem=xrecv.at[j - 1],
            device_id=(p,),
            device_id_type=pl.DeviceIdType.MESH,
        )
        rx.start()
        rdmas.append((rx, rw))

    cpx = pltpu.make_async_copy(xb_ref.at[pl.ds(r * MT, MT), :], xblk, cpxsem)
    cpx.start()
    cpw = pltpu.make_async_copy(wb_ref.at[:, pl.ds(c * NT, NT)], wblk, cpwsem)
    cpw.start()
    cpx.wait()
    cpw.wait()
    y[...] = jnp.dot(xblk[...], wblk[...], preferred_element_type=jnp.float32)

    for j in range(1, N_DEV):
        s = (d - j) % N_DEV
        rdmas[j - 1][0].wait_recv()
        cpx = pltpu.make_async_copy(xg.at[:, pl.ds(s * KS, KS)], xblk, cpxsem)
        cpx.start()
        rdmas[j - 1][1].wait_recv()
        cpw = pltpu.make_async_copy(wg.at[pl.ds(s * KS, KS), :], wblk, cpwsem)
        cpw.start()
        cpx.wait()
        cpw.wait()
        y[...] += jnp.dot(
            xblk[...], wblk[...], preferred_element_type=jnp.float32
        )

    amax_local = jnp.maximum(jnp.max(y[...]), 0.0)
    amaxbuf[pl.ds(d, 1), :] = jnp.full((1, 128), amax_local, jnp.float32)
    ardmas = []
    for j in range(1, N_DEV):
        p = (d + j) % N_DEV
        ra = pltpu.make_async_remote_copy(
            src_ref=amaxbuf.at[pl.ds(d, 1), :],
            dst_ref=amaxbuf.at[pl.ds(d, 1), :],
            send_sem=asend.at[j - 1],
            recv_sem=arecv.at[j - 1],
            device_id=(p,),
            device_id_type=pl.DeviceIdType.MESH,
        )
        ra.start()
        ardmas.append(ra)
    for ra in ardmas:
        ra.wait_recv()

    amax_g = jnp.max(amaxbuf[...])
    scale = jnp.maximum(amax_g, 1e-30) / 448.0

    qmine[...] = (jnp.maximum(y[...], 0.0) / scale).astype(jnp.float8_e4m3fn)
    qrdmas = []
    for j in range(1, N_DEV):
        p = (d + j) % N_DEV
        rq = pltpu.make_async_remote_copy(
            src_ref=qmine,
            dst_ref=qg.at[pl.ds(r * MT, MT), pl.ds(c * NT, NT)],
            send_sem=qsend.at[j - 1],
            recv_sem=qrecv.at[j - 1],
            device_id=(p,),
            device_id_type=pl.DeviceIdType.MESH,
        )
        rq.start()
        qrdmas.append(rq)

    out_copies = []
    idx = 0
    for j in range(0, N_DEV):
        s = (d - j) % N_DEV
        rs = s // PN
        cs = s % PN
        if j == 0:
            qsrc = qmine
        else:
            qrdmas[j - 1].wait_recv()
            cpq = pltpu.make_async_copy(
                qg.at[pl.ds(rs * MT, MT), pl.ds(cs * NT, NT)], qblk, cpxsem
            )
            cpq.start()
            cpq.wait()
            qsrc = qblk
        for h in range(2):
            slot = idx % 2
            if idx >= 2:
                out_copies[idx - 2].wait()
            stage[slot, :, :] = (
                qsrc[:, h * HT : (h + 1) * HT].astype(jnp.float32) * scale
            ).astype(jnp.bfloat16)
            ocp = pltpu.make_async_copy(
                stage.at[slot],
                out_ref.at[pl.ds(rs * MT, MT), pl.ds(cs * NT + h * HT, HT)],
                stsem.at[slot],
            )
            ocp.start()
            out_copies.append(ocp)
            idx += 1
    out_copies[-2].wait()
    out_copies[-1].wait()

    for rx, rw in rdmas:
        rx.wait_send()
        rw.wait_send()
    for ra in ardmas:
        ra.wait_send()
    for rq in qrdmas:
        rq.wait_send()


def kernel(x, w_mat):
    xb = x.astype(jnp.bfloat16)
    wb = w_mat.astype(jnp.bfloat16)

    out, _y, _xg, _wg, _qg = pl.pallas_call(
        _fused_body,
        out_shape=[
            jax.ShapeDtypeStruct((M, N), jnp.bfloat16),
            jax.ShapeDtypeStruct((MT, NT), jnp.float32),
            jax.ShapeDtypeStruct((MT, K), jnp.bfloat16),
            jax.ShapeDtypeStruct((K, NT), jnp.bfloat16),
            jax.ShapeDtypeStruct((M, N), jnp.float8_e4m3fn),
        ],
        in_specs=[
            pl.BlockSpec(memory_space=pl.ANY),
            pl.BlockSpec(memory_space=pl.ANY),
        ],
        out_specs=[
            pl.BlockSpec(memory_space=pl.ANY),
            pl.BlockSpec(memory_space=pltpu.VMEM),
            pl.BlockSpec(memory_space=pl.ANY),
            pl.BlockSpec(memory_space=pl.ANY),
            pl.BlockSpec(memory_space=pl.ANY),
        ],
        scratch_shapes=[
            pltpu.VMEM((MT, KS), jnp.bfloat16),
            pltpu.VMEM((KS, NT), jnp.bfloat16),
            pltpu.VMEM((MT, NT), jnp.float8_e4m3fn),
            pltpu.VMEM((MT, NT), jnp.float8_e4m3fn),
            pltpu.VMEM((2, MT, HT), jnp.bfloat16),
            pltpu.VMEM((N_DEV, 128), jnp.float32),
            pltpu.SemaphoreType.DMA,
            pltpu.SemaphoreType.DMA,
            pltpu.SemaphoreType.DMA((2,)),
            pltpu.SemaphoreType.DMA((N_DEV - 1,)),
            pltpu.SemaphoreType.DMA((N_DEV - 1,)),
            pltpu.SemaphoreType.DMA((N_DEV - 1,)),
            pltpu.SemaphoreType.DMA((N_DEV - 1,)),
            pltpu.SemaphoreType.DMA((N_DEV - 1,)),
            pltpu.SemaphoreType.DMA((N_DEV - 1,)),
            pltpu.SemaphoreType.DMA((N_DEV - 1,)),
            pltpu.SemaphoreType.DMA((N_DEV - 1,)),
        ],
        compiler_params=pltpu.CompilerParams(
            collective_id=0, vmem_limit_bytes=63 * 1024 * 1024
        ),
    )(xb, wb)
    return out
